# baseline (speedup 1.0000x reference)
import functools

import jax
import jax.numpy as jnp
from jax import lax
from jax.experimental import pallas as pl
from jax.experimental.pallas import tpu as pltpu
from jax.experimental.pallas import tpu_sc as plsc

_B = 16384
_D = 64
_L = 16

_info = plsc.get_sparse_core_info()
_NC, _NS = _info.num_cores, _info.num_subcores
_NW = _NC * _NS
_BPW = _B // _NW
_IDX_CHUNK = 128
_NCHUNK = _BPW // _IDX_CHUNK
_GROUPS = _BPW // _L


def _pooler_body(u_hbm, i_hbm, uidx_hbm, iidx_hbm, ubias_hbm, ibias_hbm,
                 gb_hbm, out_hbm,
                 uidx_v, iidx_v, ub_v, ib_v, u_v, i_v, out_v, gb_v,
                 hbuf, abuf, sem):
    wid = lax.axis_index("s") * _NC + lax.axis_index("c")
    base = wid * _BPW

    pltpu.sync_copy(uidx_hbm.at[pl.ds(base, _BPW)], uidx_v)
    pltpu.sync_copy(iidx_hbm.at[pl.ds(base, _BPW)], iidx_v)
    pltpu.sync_copy(gb_hbm, gb_v)

    copies = []
    for j in range(_NCHUNK):
        s = pl.ds(j * _IDX_CHUNK, _IDX_CHUNK)
        copies.append(pltpu.async_copy(
            ubias_hbm.at[uidx_v.at[s]], ub_v.at[s], sem))
        copies.append(pltpu.async_copy(
            ibias_hbm.at[iidx_v.at[s]], ib_v.at[s], sem))
    copies.append(pltpu.async_copy(
        u_hbm.at[pl.ds(base * _D, _BPW * _D)], u_v, sem))
    copies.append(pltpu.async_copy(
        i_hbm.at[pl.ds(base * _D, _BPW * _D)], i_v, sem))
    for c in copies:
        c.wait()

    gb = gb_v[...]
    lane = lax.iota(jnp.int32, _L)

    def group(g, _):
        row0 = g * _L
        for r in range(_L):
            rowb = (row0 + r) * _D
            p = u_v[pl.ds(rowb, _L)] * i_v[pl.ds(rowb, _L)]
            for c in range(1, _D // _L):
                p += (u_v[pl.ds(rowb + c * _L, _L)]
                      * i_v[pl.ds(rowb + c * _L, _L)])
            h = p
            rb = r * 2 * _L
            for m in (8, 4, 2, 1):
                hbuf[pl.ds(rb, _L)] = h
                h = h + hbuf[pl.ds(rb + m, _L)]
            abuf[pl.ds(r, _L)] = h
        dots = abuf[pl.ds(0, _L)]
        out_v[pl.ds(row0, _L)] = (dots + gb + ub_v[pl.ds(row0, _L)]
                                  + ib_v[pl.ds(row0, _L)])
        return _

    lax.fori_loop(0, _GROUPS, group, None, unroll=False)
    pltpu.sync_copy(out_v, out_hbm.at[pl.ds(base, _BPW)])


@jax.jit
def _pooler(u_emb, i_emb, u_idx, i_idx, ubias, ibias, gb16):
    mesh = plsc.VectorSubcoreMesh(core_axis_name="c", subcore_axis_name="s")
    f = functools.partial(
        pl.kernel, mesh=mesh,
        out_type=jax.ShapeDtypeStruct((_B,), jnp.float32),
        scratch_types=[
            pltpu.VMEM((_BPW,), jnp.int32),
            pltpu.VMEM((_BPW,), jnp.int32),
            pltpu.VMEM((_BPW,), jnp.float32),
            pltpu.VMEM((_BPW,), jnp.float32),
            pltpu.VMEM((_BPW * _D,), jnp.float32),
            pltpu.VMEM((_BPW * _D,), jnp.float32),
            pltpu.VMEM((_BPW,), jnp.float32),
            pltpu.VMEM((_L,), jnp.float32),
            pltpu.VMEM((_L * 2 * _L + _L,), jnp.float32),
            pltpu.VMEM((2 * _L,), jnp.float32),
            pltpu.SemaphoreType.DMA,
        ],
    )(_pooler_body)
    return f(u_emb, i_emb, u_idx, i_idx, ubias, ibias, gb16)


def kernel(u_emb, i_emb, u_idx, i_idx, user_bias, item_bias, global_bias):
    return _pooler(
        u_emb.reshape(-1), i_emb.reshape(-1),
        u_idx.astype(jnp.int32), i_idx.astype(jnp.int32),
        user_bias.reshape(-1), item_bias.reshape(-1),
        jnp.broadcast_to(global_bias.astype(jnp.float32), (_L,)))


# speedup vs baseline: 1.4720x; 1.4720x over previous
import functools

import jax
import jax.numpy as jnp
from jax import lax
from jax.experimental import pallas as pl
from jax.experimental.pallas import tpu as pltpu
from jax.experimental.pallas import tpu_sc as plsc

_B = 16384
_D = 64
_L = 16

_info = plsc.get_sparse_core_info()
_NC, _NS = _info.num_cores, _info.num_subcores
_NW = _NC * _NS
_BPW = _B // _NW
_IDX_CHUNK = 128
_NCHUNK = _BPW // _IDX_CHUNK
_GROUPS = _BPW // _L


def _gather_body(uidx_hbm, iidx_hbm, ubias_hbm, ibias_hbm, gb_hbm, out_hbm,
                 uidx_v, iidx_v, ub_v, ib_v, out_v, gb_v, sem):
    wid = lax.axis_index("s") * _NC + lax.axis_index("c")
    base = wid * _BPW

    pltpu.sync_copy(uidx_hbm.at[pl.ds(base, _BPW)], uidx_v)
    pltpu.sync_copy(iidx_hbm.at[pl.ds(base, _BPW)], iidx_v)
    pltpu.sync_copy(gb_hbm, gb_v)

    copies = []
    for j in range(_NCHUNK):
        s = pl.ds(j * _IDX_CHUNK, _IDX_CHUNK)
        copies.append(pltpu.async_copy(
            ubias_hbm.at[uidx_v.at[s]], ub_v.at[s], sem))
        copies.append(pltpu.async_copy(
            ibias_hbm.at[iidx_v.at[s]], ib_v.at[s], sem))
    for c in copies:
        c.wait()

    gb = gb_v[...]

    def group(g, _):
        row0 = g * _L
        out_v[pl.ds(row0, _L)] = (ub_v[pl.ds(row0, _L)]
                                  + ib_v[pl.ds(row0, _L)] + gb)
        return _

    lax.fori_loop(0, _GROUPS, group, None, unroll=True)
    pltpu.sync_copy(out_v, out_hbm.at[pl.ds(base, _BPW)])


def _dot_body(u_ref, i_ref, o_ref):
    o_ref[...] = jnp.sum(u_ref[...] * i_ref[...], axis=1)


def _add_body(a_ref, b_ref, o_ref):
    o_ref[...] = a_ref[...] + b_ref[...]


_DOT_BLK = 2048


@jax.jit
def _pooler(u_emb, i_emb, u_idx, i_idx, ubias, ibias, gb16):
    mesh = plsc.VectorSubcoreMesh(core_axis_name="c", subcore_axis_name="s")
    bias_sum = functools.partial(
        pl.kernel, mesh=mesh,
        out_type=jax.ShapeDtypeStruct((_B,), jnp.float32),
        scratch_types=[
            pltpu.VMEM((_BPW,), jnp.int32),
            pltpu.VMEM((_BPW,), jnp.int32),
            pltpu.VMEM((_BPW,), jnp.float32),
            pltpu.VMEM((_BPW,), jnp.float32),
            pltpu.VMEM((_BPW,), jnp.float32),
            pltpu.VMEM((_L,), jnp.float32),
            pltpu.SemaphoreType.DMA,
        ],
    )(_gather_body)(u_idx, i_idx, ubias, ibias, gb16)

    dot = pl.pallas_call(
        _dot_body,
        grid=(_B // _DOT_BLK,),
        in_specs=[
            pl.BlockSpec((_DOT_BLK, _D), lambda j: (j, 0)),
            pl.BlockSpec((_DOT_BLK, _D), lambda j: (j, 0)),
        ],
        out_specs=pl.BlockSpec((_DOT_BLK,), lambda j: (j,)),
        out_shape=jax.ShapeDtypeStruct((_B,), jnp.float32),
    )(u_emb, i_emb)

    return pl.pallas_call(
        _add_body,
        out_shape=jax.ShapeDtypeStruct((_B,), jnp.float32),
    )(dot, bias_sum)


def kernel(u_emb, i_emb, u_idx, i_idx, user_bias, item_bias, global_bias):
    return _pooler(
        u_emb, i_emb,
        u_idx.astype(jnp.int32), i_idx.astype(jnp.int32),
        user_bias.reshape(-1), item_bias.reshape(-1),
        jnp.broadcast_to(global_bias.astype(jnp.float32), (_L,)))


# speedup vs baseline: 2.3010x; 1.5632x over previous
import functools

import jax
import jax.numpy as jnp
from jax import lax
from jax.experimental import pallas as pl
from jax.experimental.pallas import tpu as pltpu
from jax.experimental.pallas import tpu_sc as plsc

_B = 16384
_D = 64
_L = 16

_info = plsc.get_sparse_core_info()
_NC, _NS = _info.num_cores, _info.num_subcores
_NW = _NC * _NS
_BPW = _B // _NW
_IDX_CHUNK = 128
_NCHUNK = _BPW // _IDX_CHUNK
_GROUPS = _BPW // _L


def _gather_body(uidx_hbm, iidx_hbm, ubias_hbm, ibias_hbm, gb_hbm, out_hbm,
                 uidx_v, iidx_v, ub_v, ib_v, out_v, gb_v, sem):
    wid = lax.axis_index("s") * _NC + lax.axis_index("c")
    base = wid * _BPW

    pltpu.sync_copy(uidx_hbm.at[pl.ds(base, _BPW)], uidx_v)
    pltpu.sync_copy(iidx_hbm.at[pl.ds(base, _BPW)], iidx_v)
    pltpu.sync_copy(gb_hbm, gb_v)

    copies = []
    for j in range(_NCHUNK):
        s = pl.ds(j * _IDX_CHUNK, _IDX_CHUNK)
        copies.append(pltpu.async_copy(
            ubias_hbm.at[uidx_v.at[s]], ub_v.at[s], sem))
        copies.append(pltpu.async_copy(
            ibias_hbm.at[iidx_v.at[s]], ib_v.at[s], sem))
    for c in copies:
        c.wait()

    gb = gb_v[...]

    def group(g, _):
        row0 = g * _L
        out_v[pl.ds(row0, _L)] = (ub_v[pl.ds(row0, _L)]
                                  + ib_v[pl.ds(row0, _L)] + gb)
        return _

    lax.fori_loop(0, _GROUPS, group, None, unroll=True)
    pltpu.sync_copy(out_v, out_hbm.at[pl.ds(base, _BPW)])


def _dot_body(u_ref, i_ref, o_ref):
    o_ref[...] = jnp.sum(u_ref[...] * i_ref[...], axis=0)


def _add_body(a_ref, b_ref, o_ref):
    o_ref[...] = a_ref[...] + b_ref[...]


_DOT_BLK = 2048


@jax.jit
def _pooler(u_emb, i_emb, u_idx, i_idx, ubias, ibias, gb16):
    mesh = plsc.VectorSubcoreMesh(core_axis_name="c", subcore_axis_name="s")
    bias_sum = functools.partial(
        pl.kernel, mesh=mesh,
        out_type=jax.ShapeDtypeStruct((_B,), jnp.float32),
        scratch_types=[
            pltpu.VMEM((_BPW,), jnp.int32),
            pltpu.VMEM((_BPW,), jnp.int32),
            pltpu.VMEM((_BPW,), jnp.float32),
            pltpu.VMEM((_BPW,), jnp.float32),
            pltpu.VMEM((_BPW,), jnp.float32),
            pltpu.VMEM((_L,), jnp.float32),
            pltpu.SemaphoreType.DMA,
        ],
    )(_gather_body)(u_idx, i_idx, ubias, ibias, gb16)

    dot = pl.pallas_call(
        _dot_body,
        grid=(_B // _DOT_BLK,),
        in_specs=[
            pl.BlockSpec((_D, _DOT_BLK), lambda j: (0, j)),
            pl.BlockSpec((_D, _DOT_BLK), lambda j: (0, j)),
        ],
        out_specs=pl.BlockSpec((_DOT_BLK,), lambda j: (j,)),
        out_shape=jax.ShapeDtypeStruct((_B,), jnp.float32),
    )(u_emb, i_emb)

    return pl.pallas_call(
        _add_body,
        out_shape=jax.ShapeDtypeStruct((_B,), jnp.float32),
    )(dot, bias_sum)


def kernel(u_emb, i_emb, u_idx, i_idx, user_bias, item_bias, global_bias):
    return _pooler(
        u_emb.T, i_emb.T,
        u_idx.astype(jnp.int32), i_idx.astype(jnp.int32),
        user_bias.T.reshape(-1), item_bias.T.reshape(-1),
        jnp.broadcast_to(global_bias.astype(jnp.float32), (_L,)))


# speedup vs baseline: 2.3127x; 1.0051x over previous
import functools

import jax
import jax.numpy as jnp
from jax import lax
from jax.experimental import pallas as pl
from jax.experimental.pallas import tpu as pltpu
from jax.experimental.pallas import tpu_sc as plsc

_B = 16384
_D = 64
_L = 16

_info = plsc.get_sparse_core_info()
_NC, _NS = _info.num_cores, _info.num_subcores
_NW = _NC * _NS
_BPW = _B // _NW
_IDX_CHUNK = 128
_NCHUNK = _BPW // _IDX_CHUNK
_GROUPS = _BPW // _L


def _gather_body(uidx_hbm, iidx_hbm, ubias_hbm, ibias_hbm, gb_hbm, out_hbm,
                 uidx_v, iidx_v, ub_v, ib_v, out_v, gb_v, sem):
    wid = lax.axis_index("s") * _NC + lax.axis_index("c")
    base = wid * _BPW

    pltpu.sync_copy(uidx_hbm.at[pl.ds(base, _BPW)], uidx_v)
    pltpu.sync_copy(iidx_hbm.at[pl.ds(base, _BPW)], iidx_v)
    pltpu.sync_copy(gb_hbm, gb_v)

    copies = []
    for j in range(_NCHUNK):
        s = pl.ds(j * _IDX_CHUNK, _IDX_CHUNK)
        copies.append(pltpu.async_copy(
            ubias_hbm.at[uidx_v.at[s]], ub_v.at[s], sem))
        copies.append(pltpu.async_copy(
            ibias_hbm.at[iidx_v.at[s]], ib_v.at[s], sem))
    for c in copies:
        c.wait()

    gb = gb_v[...]

    def group(g, _):
        row0 = g * _L
        out_v[pl.ds(row0, _L)] = (ub_v[pl.ds(row0, _L)]
                                  + ib_v[pl.ds(row0, _L)] + gb)
        return _

    lax.fori_loop(0, _GROUPS, group, None, unroll=True)
    pltpu.sync_copy(out_v, out_hbm.at[pl.ds(base, _BPW)])


def _dot_body(u_ref, i_ref, o_ref):
    o_ref[...] = jnp.sum(u_ref[...] * i_ref[...], axis=0)


def _add_body(a_ref, b_ref, o_ref):
    o_ref[...] = a_ref[...] + b_ref[...]


_DOT_BLK = 2048


@jax.jit
def _pooler(u_emb, i_emb, u_idx, i_idx, ubias, ibias, gb16):
    mesh = plsc.VectorSubcoreMesh(core_axis_name="c", subcore_axis_name="s")
    bias_sum = functools.partial(
        pl.kernel, mesh=mesh,
        out_type=jax.ShapeDtypeStruct((_B,), jnp.float32),
        scratch_types=[
            pltpu.VMEM((_BPW,), jnp.int32),
            pltpu.VMEM((_BPW,), jnp.int32),
            pltpu.VMEM((_BPW,), jnp.float32),
            pltpu.VMEM((_BPW,), jnp.float32),
            pltpu.VMEM((_BPW,), jnp.float32),
            pltpu.VMEM((_L,), jnp.float32),
            pltpu.SemaphoreType.DMA,
        ],
    )(_gather_body)(u_idx, i_idx, ubias, ibias, gb16)

    dot = pl.pallas_call(
        _dot_body,
        grid=(_B // _DOT_BLK,),
        in_specs=[
            pl.BlockSpec((_D, _DOT_BLK), lambda j: (0, j)),
            pl.BlockSpec((_D, _DOT_BLK), lambda j: (0, j)),
        ],
        out_specs=pl.BlockSpec((_DOT_BLK,), lambda j: (j,)),
        out_shape=jax.ShapeDtypeStruct((_B,), jnp.float32),
    )(u_emb, i_emb)

    return pl.pallas_call(
        _add_body,
        out_shape=jax.ShapeDtypeStruct((_B,), jnp.float32),
    )(dot, bias_sum)


def kernel(u_emb, i_emb, u_idx, i_idx, user_bias, item_bias, global_bias):
    return _pooler(
        u_emb.T, i_emb.T,
        u_idx.astype(jnp.int32), i_idx.astype(jnp.int32),
        user_bias.T[0], item_bias.T[0],
        jnp.broadcast_to(global_bias.astype(jnp.float32), (_L,)))


# speedup vs baseline: 2.6794x; 1.1585x over previous
import functools

import jax
import jax.numpy as jnp
from jax import lax
from jax.experimental import pallas as pl
from jax.experimental.pallas import tpu as pltpu
from jax.experimental.pallas import tpu_sc as plsc

_B = 16384
_D = 64
_L = 16

_info = plsc.get_sparse_core_info()
_NC, _NS = _info.num_cores, _info.num_subcores
_NW = _NC * _NS
_BPW = _B // _NW
_IDX_CHUNK = 128
_NCHUNK = _BPW // _IDX_CHUNK
_GROUPS = _BPW // _L


def _gather_body(uidx_hbm, iidx_hbm, ubias_hbm, ibias_hbm, gb_hbm, out_hbm,
                 uidx_v, iidx_v, ub_v, ib_v, out_v, gb_v, sem):
    wid = lax.axis_index("s") * _NC + lax.axis_index("c")
    base = wid * _BPW

    pltpu.sync_copy(uidx_hbm.at[pl.ds(base, _BPW)], uidx_v)
    pltpu.sync_copy(iidx_hbm.at[pl.ds(base, _BPW)], iidx_v)
    pltpu.sync_copy(gb_hbm, gb_v)

    copies = []
    for j in range(_NCHUNK):
        s = pl.ds(j * _IDX_CHUNK, _IDX_CHUNK)
        copies.append(pltpu.async_copy(
            ubias_hbm.at[0].at[uidx_v.at[s]], ub_v.at[s], sem))
        copies.append(pltpu.async_copy(
            ibias_hbm.at[0].at[iidx_v.at[s]], ib_v.at[s], sem))
    for c in copies:
        c.wait()

    gb = gb_v[...]

    def group(g, _):
        row0 = g * _L
        out_v[pl.ds(row0, _L)] = (ub_v[pl.ds(row0, _L)]
                                  + ib_v[pl.ds(row0, _L)] + gb)
        return _

    lax.fori_loop(0, _GROUPS, group, None, unroll=True)
    pltpu.sync_copy(out_v, out_hbm.at[pl.ds(base, _BPW)])


def _dot_body(u_ref, i_ref, o_ref):
    o_ref[...] = jnp.sum(u_ref[...] * i_ref[...], axis=0)


def _add_body(a_ref, b_ref, o_ref):
    o_ref[...] = a_ref[...] + b_ref[...]


_DOT_BLK = 2048


@jax.jit
def _pooler(u_emb, i_emb, u_idx, i_idx, ubias, ibias, gb16):
    mesh = plsc.VectorSubcoreMesh(core_axis_name="c", subcore_axis_name="s")
    bias_sum = functools.partial(
        pl.kernel, mesh=mesh,
        out_type=jax.ShapeDtypeStruct((_B,), jnp.float32),
        scratch_types=[
            pltpu.VMEM((_BPW,), jnp.int32),
            pltpu.VMEM((_BPW,), jnp.int32),
            pltpu.VMEM((_BPW,), jnp.float32),
            pltpu.VMEM((_BPW,), jnp.float32),
            pltpu.VMEM((_BPW,), jnp.float32),
            pltpu.VMEM((_L,), jnp.float32),
            pltpu.SemaphoreType.DMA,
        ],
    )(_gather_body)(u_idx, i_idx, ubias, ibias, gb16)

    dot = pl.pallas_call(
        _dot_body,
        grid=(_B // _DOT_BLK,),
        in_specs=[
            pl.BlockSpec((_D, _DOT_BLK), lambda j: (0, j)),
            pl.BlockSpec((_D, _DOT_BLK), lambda j: (0, j)),
        ],
        out_specs=pl.BlockSpec((_DOT_BLK,), lambda j: (j,)),
        out_shape=jax.ShapeDtypeStruct((_B,), jnp.float32),
    )(u_emb, i_emb)

    return pl.pallas_call(
        _add_body,
        out_shape=jax.ShapeDtypeStruct((_B,), jnp.float32),
    )(dot, bias_sum)


def kernel(u_emb, i_emb, u_idx, i_idx, user_bias, item_bias, global_bias):
    return _pooler(
        u_emb.T, i_emb.T,
        u_idx.astype(jnp.int32), i_idx.astype(jnp.int32),
        user_bias.T, item_bias.T,
        jnp.broadcast_to(global_bias.astype(jnp.float32), (_L,)))


# speedup vs baseline: 2.8180x; 1.0518x over previous
import functools

import jax
import jax.numpy as jnp
from jax import lax
from jax.experimental import pallas as pl
from jax.experimental.pallas import tpu as pltpu
from jax.experimental.pallas import tpu_sc as plsc

_B = 16384
_D = 64
_L = 16

_info = plsc.get_sparse_core_info()
_NC, _NS = _info.num_cores, _info.num_subcores
_NW = _NC * _NS
_BPW = _B // _NW
_IDX_CHUNK = 128
_NCHUNK = _BPW // _IDX_CHUNK
_GROUPS = _BPW // _L


def _gather_body(uidx_hbm, iidx_hbm, ubias_hbm, ibias_hbm, out_hbm,
                 uidx_v, iidx_v, ub_v, ib_v, out_v, isem, sem):
    wid = lax.axis_index("s") * _NC + lax.axis_index("c")
    base = wid * _BPW

    cu = pltpu.async_copy(uidx_hbm.at[pl.ds(base, _BPW)], uidx_v, isem)
    ci = pltpu.async_copy(iidx_hbm.at[pl.ds(base, _BPW)], iidx_v, isem)

    copies = []
    cu.wait()
    for j in range(_NCHUNK):
        s = pl.ds(j * _IDX_CHUNK, _IDX_CHUNK)
        copies.append(pltpu.async_copy(
            ubias_hbm.at[0].at[uidx_v.at[s]], ub_v.at[s], sem))
    ci.wait()
    for j in range(_NCHUNK):
        s = pl.ds(j * _IDX_CHUNK, _IDX_CHUNK)
        copies.append(pltpu.async_copy(
            ibias_hbm.at[0].at[iidx_v.at[s]], ib_v.at[s], sem))
    for c in copies:
        c.wait()

    def group(g, _):
        row0 = g * _L
        out_v[pl.ds(row0, _L)] = (ub_v[pl.ds(row0, _L)]
                                  + ib_v[pl.ds(row0, _L)])
        return _

    lax.fori_loop(0, _GROUPS, group, None, unroll=True)
    pltpu.sync_copy(out_v, out_hbm.at[pl.ds(base, _BPW)])


def _dot_body(u_ref, i_ref, o_ref):
    o_ref[...] = jnp.sum(u_ref[...] * i_ref[...], axis=0)


def _add_body(g_ref, a_ref, b_ref, o_ref):
    o_ref[...] = a_ref[...] + b_ref[...] + g_ref[0]


_DOT_BLK = 2048


@jax.jit
def _pooler(u_emb, i_emb, u_idx, i_idx, ubias, ibias, gb):
    mesh = plsc.VectorSubcoreMesh(core_axis_name="c", subcore_axis_name="s")
    bias_sum = functools.partial(
        pl.kernel, mesh=mesh,
        out_type=jax.ShapeDtypeStruct((_B,), jnp.float32),
        scratch_types=[
            pltpu.VMEM((_BPW,), jnp.int32),
            pltpu.VMEM((_BPW,), jnp.int32),
            pltpu.VMEM((_BPW,), jnp.float32),
            pltpu.VMEM((_BPW,), jnp.float32),
            pltpu.VMEM((_BPW,), jnp.float32),
            pltpu.SemaphoreType.DMA,
            pltpu.SemaphoreType.DMA,
        ],
    )(_gather_body)(u_idx, i_idx, ubias, ibias)

    dot = pl.pallas_call(
        _dot_body,
        grid=(_B // _DOT_BLK,),
        in_specs=[
            pl.BlockSpec((_D, _DOT_BLK), lambda j: (0, j)),
            pl.BlockSpec((_D, _DOT_BLK), lambda j: (0, j)),
        ],
        out_specs=pl.BlockSpec((_DOT_BLK,), lambda j: (j,)),
        out_shape=jax.ShapeDtypeStruct((_B,), jnp.float32),
    )(u_emb, i_emb)

    return pl.pallas_call(
        _add_body,
        in_specs=[
            pl.BlockSpec(memory_space=pltpu.SMEM),
            pl.BlockSpec((_B,), lambda: (0,)),
            pl.BlockSpec((_B,), lambda: (0,)),
        ],
        out_shape=jax.ShapeDtypeStruct((_B,), jnp.float32),
    )(gb, dot, bias_sum)


def kernel(u_emb, i_emb, u_idx, i_idx, user_bias, item_bias, global_bias):
    return _pooler(
        u_emb.T, i_emb.T,
        u_idx.astype(jnp.int32), i_idx.astype(jnp.int32),
        user_bias.T, item_bias.T,
        global_bias.astype(jnp.float32))
